# baseline (speedup 1.0000x reference)
import functools

import jax
import jax.numpy as jnp
from jax import lax
from jax.experimental import pallas as pl
from jax.experimental.pallas import tpu as pltpu
from jax.experimental.pallas import tpu_sc as plsc

D_MODEL = 128
N_VOCAB = 5
N_WORKERS = 32
CHUNK = 128


def _fuse_table_body(emb_ref, base_ref, out_ref):
    out_ref[...] = emb_ref[...] + base_ref[...]


def _fuse_table(emb_key, base_action_emb):
    return pl.pallas_call(
        _fuse_table_body,
        out_shape=jax.ShapeDtypeStruct((N_VOCAB, D_MODEL), jnp.float32),
    )(emb_key, base_action_emb.reshape(1, D_MODEL))


def _sc_lookup_body(table_hbm, idx_hbm, out_hbm, idx_v, rows_v, sem):
    n_total = idx_hbm.shape[0]
    n_per_w = n_total // N_WORKERS
    n_chunks = n_per_w // CHUNK
    wid = lax.axis_index("s") * 2 + lax.axis_index("c")
    base = wid * n_per_w

    def body(i, _):
        off = base + i * CHUNK
        pltpu.sync_copy(idx_hbm.at[pl.ds(off, CHUNK)], idx_v)
        pltpu.async_copy(table_hbm.at[idx_v], rows_v, sem).wait()
        pltpu.sync_copy(rows_v, out_hbm.at[pl.ds(off, CHUNK)])
        return 0

    lax.fori_loop(0, n_chunks, body, 0)


def _sc_lookup(table, idx_flat):
    n_total = idx_flat.shape[0]
    mesh = plsc.VectorSubcoreMesh(core_axis_name="c", subcore_axis_name="s")
    f = functools.partial(
        pl.kernel,
        mesh=mesh,
        out_type=jax.ShapeDtypeStruct((n_total, D_MODEL), jnp.float32),
        scratch_types=[
            pltpu.VMEM((CHUNK,), jnp.int32),
            pltpu.VMEM((CHUNK, D_MODEL), jnp.float32),
            pltpu.SemaphoreType.DMA,
        ],
    )(_sc_lookup_body)
    return f(table, idx_flat)


def kernel(actions, emb_key, base_action_emb):
    B, T = actions.shape
    fused = _fuse_table(emb_key, base_action_emb)
    idx_flat = actions.reshape(-1).astype(jnp.int32)
    out_flat = _sc_lookup(fused, idx_flat)
    return out_flat.reshape(B, T, 1, D_MODEL)
